# Initial kernel scaffold; baseline (speedup 1.0000x reference)
import jax
import jax.numpy as jnp
from jax.experimental import pallas as pl


def kernel(x_user, x_item, edge_index_ut, edge_index_tu, W_user, b_user, W_item, b_item, att_src_ut, att_dst_ut, att_src_tu, att_dst_tu, k_W, k_b, q):
    raise NotImplementedError("write your pallas kernel here")


# speedup vs baseline: 24.0206x; 24.0206x over previous
import functools

import jax
import jax.numpy as jnp
from jax import lax
from jax.experimental import pallas as pl
from jax.experimental.pallas import tpu as pltpu
from jax.experimental.pallas import tpu_sc as plsc

N_NODE = 10000
N_EDGE = 320000
D = 128
NH = 8
DH = 16
ACC_W = 144

N_SUBCORES = 16
EDGES_PER_TILE = N_EDGE // N_SUBCORES
CHUNK = 80
N_CHUNKS = EDGES_PER_TILE // CHUNK
HALF = 5008
DUMMY = HALF
ACC_ROWS = 5024
ZROWS = 112
TBL_ROWS_PER_TILE = 2 * N_NODE // N_SUBCORES
SWEEP_ROWS = (HALF, N_NODE - HALF)


def _proj_body(x_ref, w_ref, b_ref, a_ref, h_ref, al_ref):
    h = jnp.dot(x_ref[...], w_ref[...], preferred_element_type=jnp.float32)
    h = h + b_ref[...]
    h_ref[...] = h
    al_ref[...] = jnp.dot(h, a_ref[...], preferred_element_type=jnp.float32)


def _project(x, w, b, a):
    blk = 1000
    grid = (N_NODE // blk,)
    return pl.pallas_call(
        _proj_body,
        grid=grid,
        in_specs=[
            pl.BlockSpec((blk, D), lambda i: (i, 0)),
            pl.BlockSpec((D, D), lambda i: (0, 0)),
            pl.BlockSpec((1, D), lambda i: (0, 0)),
            pl.BlockSpec((D, 32), lambda i: (0, 0)),
        ],
        out_specs=[
            pl.BlockSpec((blk, D), lambda i: (i, 0)),
            pl.BlockSpec((blk, 32), lambda i: (i, 0)),
        ],
        out_shape=[
            jax.ShapeDtypeStruct((N_NODE, D), jnp.float32),
            jax.ShapeDtypeStruct((N_NODE, 32), jnp.float32),
        ],
    )(x, w, b.reshape(1, D), a)


def _sc_edge_kernel(h_cat, table_cat, src1d, dst1d, raw,
                    zbuf_v, tbuf_v, h_v, msg_v, as_v, ad_v,
                    idx_src, idx_dst, idx_srcp, idx_dstd, idx_dstt,
                    table_sp, acc_sp, sem_a, sem_b, sem_c):
    c = lax.axis_index("c")
    s = lax.axis_index("s")

    zero16 = jnp.zeros((DH,), jnp.float32)
    head_idx = [jnp.full((DH,), h, jnp.int32) for h in range(NH)]

    def _zero_row(r, _):
        for j in range(ACC_W // DH):
            zbuf_v[r, pl.ds(j * DH, DH)] = zero16
        return 0
    lax.fori_loop(0, ZROWS, _zero_row, 0)

    trow = s * TBL_ROWS_PER_TILE
    pltpu.sync_copy(table_cat.at[pl.ds(c * 2 * N_NODE + trow, TBL_ROWS_PER_TILE)], tbuf_v)
    pltpu.sync_copy(tbuf_v, table_sp.at[pl.ds(trow, TBL_ROWS_PER_TILE)])

    ebase0 = c * N_EDGE + s * EDGES_PER_TILE
    c10k = c * N_NODE

    for p in range(2):
        z0 = s * (ACC_ROWS // N_SUBCORES)
        for zr, zn in ((0, ZROWS), (ZROWS, ZROWS), (2 * ZROWS, 90)):
            pltpu.sync_copy(zbuf_v.at[pl.ds(0, zn)], acc_sp.at[pl.ds(z0 + zr, zn)])
        plsc.subcore_barrier()

        lo = p * HALF

        def _chunk(ch, _):
            ebase = ebase0 + ch * CHUNK
            pltpu.sync_copy(src1d.at[pl.ds(ebase, CHUNK)], idx_src)
            pltpu.sync_copy(dst1d.at[pl.ds(ebase, CHUNK)], idx_dst)
            for j in range(CHUNK // DH):
                sl = pl.ds(j * DH, DH)
                idx_srcp[sl] = idx_src[sl] + c10k
                idx_dstt[sl] = idx_dst[sl] + N_NODE
                t = idx_dst[sl] - lo
                if p == 0:
                    idx_dstd[sl] = jnp.minimum(t, DUMMY)
                else:
                    idx_dstd[sl] = jnp.where(t >= 0, t, DUMMY)
            d_as = pltpu.async_copy(table_sp.at[idx_src], as_v, sem_a)
            d_ad = pltpu.async_copy(table_sp.at[idx_dstt], ad_v, sem_b)
            d_h = pltpu.async_copy(h_cat.at[idx_srcp], h_v, sem_c)
            d_as.wait()
            d_ad.wait()
            d_h.wait()

            def _edge(e, _):
                a = as_v[e, :] + ad_v[e, :]
                a = jnp.maximum(a, a * 0.2)
                ev = jnp.exp(a)
                msg_v[e, pl.ds(D, DH)] = ev
                for h in range(NH):
                    ebc = ev.at[head_idx[h]].get(mode='promise_in_bounds')
                    sl = pl.ds(h * DH, DH)
                    msg_v[e, sl] = h_v[e, sl] * ebc
                return 0
            lax.fori_loop(0, CHUNK, _edge, 0)

            pltpu.sync_copy(msg_v, acc_sp.at[idx_dstd], add=True)
            return 0
        lax.fori_loop(0, N_CHUNKS, _chunk, 0)
        plsc.subcore_barrier()

        rp = SWEEP_ROWS[p] // N_SUBCORES
        r0 = s * rp
        pltpu.sync_copy(acc_sp.at[pl.ds(r0, rp)],
                        raw.at[pl.ds(c * N_NODE + lo + r0, rp)])
        plsc.subcore_barrier()


def _sc_edge_phase(h_cat, table_cat, src1d, dst1d):
    mesh = plsc.VectorSubcoreMesh(core_axis_name="c", subcore_axis_name="s")
    run = functools.partial(
        pl.kernel,
        mesh=mesh,
        compiler_params=pltpu.CompilerParams(use_tc_tiling_on_sc=False,
                                             needs_layout_passes=False),
        out_type=jax.ShapeDtypeStruct((2 * N_NODE, ACC_W), jnp.float32),
        scratch_types=[
            pltpu.VMEM((ZROWS, ACC_W), jnp.float32),
            pltpu.VMEM((TBL_ROWS_PER_TILE, DH), jnp.float32),
            pltpu.VMEM((CHUNK, D), jnp.float32),
            pltpu.VMEM((CHUNK, ACC_W), jnp.float32),
            pltpu.VMEM((CHUNK, DH), jnp.float32),
            pltpu.VMEM((CHUNK, DH), jnp.float32),
            pltpu.VMEM((CHUNK,), jnp.int32),
            pltpu.VMEM((CHUNK,), jnp.int32),
            pltpu.VMEM((CHUNK,), jnp.int32),
            pltpu.VMEM((CHUNK,), jnp.int32),
            pltpu.VMEM((CHUNK,), jnp.int32),
            pltpu.VMEM_SHARED((2 * N_NODE, DH), jnp.float32),
            pltpu.VMEM_SHARED((ACC_ROWS, ACC_W), jnp.float32),
            pltpu.SemaphoreType.DMA,
            pltpu.SemaphoreType.DMA,
            pltpu.SemaphoreType.DMA,
        ],
    )(_sc_edge_kernel)
    return run(h_cat, table_cat, src1d, dst1d)


def _norm_body(rt_ref, raw_ref, out_ref):
    raw = raw_ref[...]
    msg = raw[:, 0:D]
    den = raw[:, D:D + NH]
    drep = jnp.dot(den, rt_ref[...], preferred_element_type=jnp.float32)
    out_ref[...] = jnp.maximum(msg / (drep + 1e-16), 0.0)


def _normalize(raw, rt):
    blk = 1000
    grid = (2 * N_NODE // blk,)
    return pl.pallas_call(
        _norm_body,
        grid=grid,
        in_specs=[
            pl.BlockSpec((NH, D), lambda i: (0, 0)),
            pl.BlockSpec((blk, ACC_W), lambda i: (i, 0)),
        ],
        out_specs=pl.BlockSpec((blk, D), lambda i: (i, 0)),
        out_shape=jax.ShapeDtypeStruct((2 * N_NODE, D), jnp.float32),
    )(rt, raw)


def kernel(x_user, x_item, edge_index_ut, edge_index_tu, W_user, b_user,
           W_item, b_item, att_src_ut, att_dst_ut, att_src_tu, att_dst_tu,
           k_W, k_b, q):
    f32 = jnp.float32
    rep = jnp.repeat(jnp.eye(NH, dtype=f32), DH, axis=0)
    z8 = jnp.zeros((D, NH), f32)

    def att_cols(att):
        return rep * att.reshape(D)[:, None]

    A_user = jnp.concatenate(
        [att_cols(att_src_ut), z8, att_cols(att_dst_tu), z8], axis=1)
    A_item = jnp.concatenate(
        [att_cols(att_src_tu), z8, att_cols(att_dst_ut), z8], axis=1)

    h_u, al_u = _project(x_user, W_user, b_user, A_user)
    h_i, al_i = _project(x_item, W_item, b_item, A_item)

    h_cat = jnp.concatenate([h_i, h_u], axis=0)
    table_tu = jnp.concatenate([al_i[:, 0:16], al_u[:, 16:32]], axis=0)
    table_ut = jnp.concatenate([al_u[:, 0:16], al_i[:, 16:32]], axis=0)
    table_cat = jnp.concatenate([table_tu, table_ut], axis=0)

    ei_tu = jnp.asarray(edge_index_tu, jnp.int32)
    ei_ut = jnp.asarray(edge_index_ut, jnp.int32)
    src1d = jnp.concatenate([ei_tu[0], ei_ut[0]])
    dst1d = jnp.concatenate([ei_tu[1], ei_ut[1]])

    raw = _sc_edge_phase(h_cat, table_cat, src1d, dst1d)
    return _normalize(raw, rep.T)


# speedup vs baseline: 50.2702x; 2.0928x over previous
import functools

import jax
import jax.numpy as jnp
from jax import lax
from jax.experimental import pallas as pl
from jax.experimental.pallas import tpu as pltpu
from jax.experimental.pallas import tpu_sc as plsc

N_NODE = 10000
N_EDGE = 320000
D = 128
NH = 8
DH = 16
ACC_W = 144

N_SUBCORES = 16
EDGES_PER_TILE = N_EDGE // N_SUBCORES
CHUNK = 80
N_CHUNKS = EDGES_PER_TILE // CHUNK
HALF = 5008
DUMMY = HALF
ACC_ROWS = 5024
ZROWS = 112
TBL_ROWS_PER_TILE = 2 * N_NODE // N_SUBCORES
SWEEP_ROWS = (HALF, N_NODE - HALF)


def _proj_body(x_ref, w_ref, b_ref, a_ref, h_ref, al_ref):
    h = jnp.dot(x_ref[...], w_ref[...], preferred_element_type=jnp.float32)
    h = h + b_ref[...]
    h_ref[...] = h
    al_ref[...] = jnp.dot(h, a_ref[...], preferred_element_type=jnp.float32)


def _project(x, w, b, a):
    blk = 1000
    grid = (N_NODE // blk,)
    return pl.pallas_call(
        _proj_body,
        grid=grid,
        in_specs=[
            pl.BlockSpec((blk, D), lambda i: (i, 0)),
            pl.BlockSpec((D, D), lambda i: (0, 0)),
            pl.BlockSpec((1, D), lambda i: (0, 0)),
            pl.BlockSpec((D, 32), lambda i: (0, 0)),
        ],
        out_specs=[
            pl.BlockSpec((blk, D), lambda i: (i, 0)),
            pl.BlockSpec((blk, 32), lambda i: (i, 0)),
        ],
        out_shape=[
            jax.ShapeDtypeStruct((N_NODE, D), jnp.float32),
            jax.ShapeDtypeStruct((N_NODE, 32), jnp.float32),
        ],
    )(x, w, b.reshape(1, D), a)


def _sc_edge_kernel(h_cat, table_cat, src1d, dst1d, raw,
                    zbuf_v, tbuf_v, h_v, msg_v, as_v, ad_v,
                    idx_src, idx_dst, idx_srcp, idx_dstd, idx_dstt,
                    table_sp, acc_sp, sem_a, sem_b, sem_c):
    c = lax.axis_index("c")
    s = lax.axis_index("s")

    zero16 = jnp.zeros((DH,), jnp.float32)
    head_idx = [jnp.full((DH,), h, jnp.int32) for h in range(NH)]

    def _zero_row(r, _):
        for j in range(ACC_W // DH):
            zbuf_v[r, pl.ds(j * DH, DH)] = zero16
        return 0
    lax.fori_loop(0, ZROWS, _zero_row, 0)

    trow = s * TBL_ROWS_PER_TILE
    pltpu.sync_copy(table_cat.at[pl.ds(c * 2 * N_NODE + trow, TBL_ROWS_PER_TILE)], tbuf_v)
    pltpu.sync_copy(tbuf_v, table_sp.at[pl.ds(trow, TBL_ROWS_PER_TILE)])

    ebase0 = c * N_EDGE + s * EDGES_PER_TILE
    c10k = c * N_NODE

    for p in range(2):
        z0 = s * (ACC_ROWS // N_SUBCORES)
        for zr, zn in ((0, ZROWS), (ZROWS, ZROWS), (2 * ZROWS, 90)):
            pltpu.sync_copy(zbuf_v.at[pl.ds(0, zn)], acc_sp.at[pl.ds(z0 + zr, zn)])
        plsc.subcore_barrier()

        lo = p * HALF

        def _chunk(ch, _):
            ebase = ebase0 + ch * CHUNK
            pltpu.sync_copy(src1d.at[pl.ds(ebase, CHUNK)], idx_src)
            pltpu.sync_copy(dst1d.at[pl.ds(ebase, CHUNK)], idx_dst)
            for j in range(CHUNK // DH):
                sl = pl.ds(j * DH, DH)
                idx_srcp[sl] = idx_src[sl] + c10k
                idx_dstt[sl] = idx_dst[sl] + N_NODE
                t = idx_dst[sl] - lo
                if p == 0:
                    idx_dstd[sl] = jnp.minimum(t, DUMMY)
                else:
                    idx_dstd[sl] = jnp.where(t >= 0, t, DUMMY)
            d_as = pltpu.async_copy(table_sp.at[idx_src], as_v, sem_a)
            d_ad = pltpu.async_copy(table_sp.at[idx_dstt], ad_v, sem_b)
            d_h = pltpu.async_copy(h_cat.at[idx_srcp], h_v, sem_c)
            d_as.wait()
            d_ad.wait()
            d_h.wait()

            @plsc.parallel_loop(0, CHUNK, unroll=4)
            def _edge(e):
                a = as_v[e, :] + ad_v[e, :]
                a = jnp.maximum(a, a * 0.2)
                ev = jnp.exp(a)
                msg_v[e, pl.ds(D, DH)] = ev
                for h in range(NH):
                    ebc = ev.at[head_idx[h]].get(mode='promise_in_bounds')
                    sl = pl.ds(h * DH, DH)
                    msg_v[e, sl] = h_v[e, sl] * ebc

            pltpu.sync_copy(msg_v, acc_sp.at[idx_dstd], add=True)
            return 0
        lax.fori_loop(0, N_CHUNKS, _chunk, 0)
        plsc.subcore_barrier()

        rp = SWEEP_ROWS[p] // N_SUBCORES
        r0 = s * rp
        pltpu.sync_copy(acc_sp.at[pl.ds(r0, rp)],
                        raw.at[pl.ds(c * N_NODE + lo + r0, rp)])
        plsc.subcore_barrier()


def _sc_edge_phase(h_cat, table_cat, src1d, dst1d):
    mesh = plsc.VectorSubcoreMesh(core_axis_name="c", subcore_axis_name="s")
    run = functools.partial(
        pl.kernel,
        mesh=mesh,
        compiler_params=pltpu.CompilerParams(use_tc_tiling_on_sc=False,
                                             needs_layout_passes=False),
        out_type=jax.ShapeDtypeStruct((2 * N_NODE, ACC_W), jnp.float32),
        scratch_types=[
            pltpu.VMEM((ZROWS, ACC_W), jnp.float32),
            pltpu.VMEM((TBL_ROWS_PER_TILE, DH), jnp.float32),
            pltpu.VMEM((CHUNK, D), jnp.float32),
            pltpu.VMEM((CHUNK, ACC_W), jnp.float32),
            pltpu.VMEM((CHUNK, DH), jnp.float32),
            pltpu.VMEM((CHUNK, DH), jnp.float32),
            pltpu.VMEM((CHUNK,), jnp.int32),
            pltpu.VMEM((CHUNK,), jnp.int32),
            pltpu.VMEM((CHUNK,), jnp.int32),
            pltpu.VMEM((CHUNK,), jnp.int32),
            pltpu.VMEM((CHUNK,), jnp.int32),
            pltpu.VMEM_SHARED((2 * N_NODE, DH), jnp.float32),
            pltpu.VMEM_SHARED((ACC_ROWS, ACC_W), jnp.float32),
            pltpu.SemaphoreType.DMA,
            pltpu.SemaphoreType.DMA,
            pltpu.SemaphoreType.DMA,
        ],
    )(_sc_edge_kernel)
    return run(h_cat, table_cat, src1d, dst1d)


def _norm_body(rt_ref, raw_ref, out_ref):
    raw = raw_ref[...]
    msg = raw[:, 0:D]
    den = raw[:, D:D + NH]
    drep = jnp.dot(den, rt_ref[...], preferred_element_type=jnp.float32)
    out_ref[...] = jnp.maximum(msg / (drep + 1e-16), 0.0)


def _normalize(raw, rt):
    blk = 1000
    grid = (2 * N_NODE // blk,)
    return pl.pallas_call(
        _norm_body,
        grid=grid,
        in_specs=[
            pl.BlockSpec((NH, D), lambda i: (0, 0)),
            pl.BlockSpec((blk, ACC_W), lambda i: (i, 0)),
        ],
        out_specs=pl.BlockSpec((blk, D), lambda i: (i, 0)),
        out_shape=jax.ShapeDtypeStruct((2 * N_NODE, D), jnp.float32),
    )(rt, raw)


def kernel(x_user, x_item, edge_index_ut, edge_index_tu, W_user, b_user,
           W_item, b_item, att_src_ut, att_dst_ut, att_src_tu, att_dst_tu,
           k_W, k_b, q):
    f32 = jnp.float32
    rep = jnp.repeat(jnp.eye(NH, dtype=f32), DH, axis=0)
    z8 = jnp.zeros((D, NH), f32)

    def att_cols(att):
        return rep * att.reshape(D)[:, None]

    A_user = jnp.concatenate(
        [att_cols(att_src_ut), z8, att_cols(att_dst_tu), z8], axis=1)
    A_item = jnp.concatenate(
        [att_cols(att_src_tu), z8, att_cols(att_dst_ut), z8], axis=1)

    h_u, al_u = _project(x_user, W_user, b_user, A_user)
    h_i, al_i = _project(x_item, W_item, b_item, A_item)

    h_cat = jnp.concatenate([h_i, h_u], axis=0)
    table_tu = jnp.concatenate([al_i[:, 0:16], al_u[:, 16:32]], axis=0)
    table_ut = jnp.concatenate([al_u[:, 0:16], al_i[:, 16:32]], axis=0)
    table_cat = jnp.concatenate([table_tu, table_ut], axis=0)

    ei_tu = jnp.asarray(edge_index_tu, jnp.int32)
    ei_ut = jnp.asarray(edge_index_ut, jnp.int32)
    src1d = jnp.concatenate([ei_tu[0], ei_ut[0]])
    dst1d = jnp.concatenate([ei_tu[1], ei_ut[1]])

    raw = _sc_edge_phase(h_cat, table_cat, src1d, dst1d)
    return _normalize(raw, rep.T)


# speedup vs baseline: 68.3771x; 1.3602x over previous
import functools

import jax
import jax.numpy as jnp
from jax import lax
from jax.experimental import pallas as pl
from jax.experimental.pallas import tpu as pltpu
from jax.experimental.pallas import tpu_sc as plsc

N_NODE = 10000
N_EDGE = 320000
D = 128
NH = 8
DH = 16
ACC_W = 144

N_SUBCORES = 16
EDGES_PER_TILE = N_EDGE // N_SUBCORES
CHUNK = 80
N_CHUNKS = EDGES_PER_TILE // CHUNK
HALF = 5008
DUMMY = HALF
ACC_ROWS = 5024
ZROWS = 112
TBL_ROWS_PER_TILE = 2 * N_NODE // N_SUBCORES
SWEEP_ROWS = (HALF, N_NODE - HALF)


def _proj_body(x_ref, w_ref, b_ref, a_ref, h_ref, al_ref):
    h = jnp.dot(x_ref[...], w_ref[...], preferred_element_type=jnp.float32)
    h = h + b_ref[...]
    h_ref[...] = h
    al_ref[...] = jnp.dot(h, a_ref[...], preferred_element_type=jnp.float32)


def _project(x, w, b, a):
    blk = 1000
    grid = (N_NODE // blk,)
    return pl.pallas_call(
        _proj_body,
        grid=grid,
        in_specs=[
            pl.BlockSpec((blk, D), lambda i: (i, 0)),
            pl.BlockSpec((D, D), lambda i: (0, 0)),
            pl.BlockSpec((1, D), lambda i: (0, 0)),
            pl.BlockSpec((D, 32), lambda i: (0, 0)),
        ],
        out_specs=[
            pl.BlockSpec((blk, D), lambda i: (i, 0)),
            pl.BlockSpec((blk, 32), lambda i: (i, 0)),
        ],
        out_shape=[
            jax.ShapeDtypeStruct((N_NODE, D), jnp.float32),
            jax.ShapeDtypeStruct((N_NODE, 32), jnp.float32),
        ],
    )(x, w, b.reshape(1, D), a)


def _sc_edge_kernel(h_cat, table_cat, src1d, dst1d, raw,
                    zbuf_v, h_v0, h_v1, msg_v, as_v0, as_v1,
                    ad_v0, ad_v1,
                    idx_src0, idx_src1, idx_dst0, idx_dst1, idx_srcp0,
                    idx_srcp1, idx_dstd0, idx_dstd1, idx_dstt0, idx_dstt1,
                    idx_srct0, idx_srct1,
                    acc_sp, sem_a0, sem_a1, sem_b0, sem_b1,
                    sem_c0, sem_c1):
    h_v = (h_v0, h_v1)
    as_v = (as_v0, as_v1)
    ad_v = (ad_v0, ad_v1)
    idx_src = (idx_src0, idx_src1)
    idx_dst = (idx_dst0, idx_dst1)
    idx_srcp = (idx_srcp0, idx_srcp1)
    idx_dstd = (idx_dstd0, idx_dstd1)
    idx_dstt = (idx_dstt0, idx_dstt1)
    idx_srct = (idx_srct0, idx_srct1)
    sem_a = (sem_a0, sem_a1)
    sem_b = (sem_b0, sem_b1)
    sem_c = (sem_c0, sem_c1)
    c = lax.axis_index("c")
    s = lax.axis_index("s")

    zero16 = jnp.zeros((DH,), jnp.float32)
    head_idx = [jnp.full((DH,), h, jnp.int32) for h in range(NH)]

    def _zero_row(r, _):
        for j in range(ACC_W // DH):
            zbuf_v[r, pl.ds(j * DH, DH)] = zero16
        return 0
    lax.fori_loop(0, ZROWS, _zero_row, 0)

    ebase0 = c * N_EDGE + s * EDGES_PER_TILE
    c10k = c * N_NODE
    ctbl = c * 2 * N_NODE

    for p in range(2):
        z0 = s * (ACC_ROWS // N_SUBCORES)
        for zr, zn in ((0, ZROWS), (ZROWS, ZROWS), (2 * ZROWS, 90)):
            pltpu.sync_copy(zbuf_v.at[pl.ds(0, zn)], acc_sp.at[pl.ds(z0 + zr, zn)])
        plsc.subcore_barrier()

        lo = p * HALF

        def _stage(ch, b):
            ebase = ebase0 + ch * CHUNK
            pltpu.sync_copy(src1d.at[pl.ds(ebase, CHUNK)], idx_src[b])
            pltpu.sync_copy(dst1d.at[pl.ds(ebase, CHUNK)], idx_dst[b])
            for j in range(CHUNK // DH):
                sl = pl.ds(j * DH, DH)
                idx_srcp[b][sl] = idx_src[b][sl] + c10k
                idx_srct[b][sl] = idx_src[b][sl] + ctbl
                idx_dstt[b][sl] = idx_dst[b][sl] + (ctbl + N_NODE)
                t = idx_dst[b][sl] - lo
                if p == 0:
                    idx_dstd[b][sl] = jnp.minimum(t, DUMMY)
                else:
                    idx_dstd[b][sl] = jnp.where(t >= 0, t, DUMMY)
            return (pltpu.async_copy(table_cat.at[idx_srct[b]], as_v[b], sem_a[b]),
                    pltpu.async_copy(table_cat.at[idx_dstt[b]], ad_v[b], sem_b[b]),
                    pltpu.async_copy(h_cat.at[idx_srcp[b]], h_v[b], sem_c[b]))

        def _process(b, dmas):
            for d in dmas:
                d.wait()

            @plsc.parallel_loop(0, CHUNK, unroll=4)
            def _edge(e):
                a = as_v[b][e, :] + ad_v[b][e, :]
                a = jnp.maximum(a, a * 0.2)
                ev = jnp.exp(a)
                msg_v[e, pl.ds(D, DH)] = ev
                for h in range(NH):
                    ebc = ev.at[head_idx[h]].get(mode='promise_in_bounds')
                    sl = pl.ds(h * DH, DH)
                    msg_v[e, sl] = h_v[b][e, sl] * ebc

            pltpu.sync_copy(msg_v, acc_sp.at[idx_dstd[b]], add=True)

        def _chunk2(ch2, _):
            d0 = _stage(2 * ch2, 0)
            d1 = _stage(2 * ch2 + 1, 1)
            _process(0, d0)
            _process(1, d1)
            return 0
        lax.fori_loop(0, N_CHUNKS // 2, _chunk2, 0)
        plsc.subcore_barrier()

        rp = SWEEP_ROWS[p] // N_SUBCORES
        r0 = s * rp
        pltpu.sync_copy(acc_sp.at[pl.ds(r0, rp)],
                        raw.at[pl.ds(c * N_NODE + lo + r0, rp)])
        plsc.subcore_barrier()


def _sc_edge_phase(h_cat, table_cat, src1d, dst1d):
    mesh = plsc.VectorSubcoreMesh(core_axis_name="c", subcore_axis_name="s")
    run = functools.partial(
        pl.kernel,
        mesh=mesh,
        compiler_params=pltpu.CompilerParams(use_tc_tiling_on_sc=False,
                                             needs_layout_passes=False),
        out_type=jax.ShapeDtypeStruct((2 * N_NODE, ACC_W), jnp.float32),
        scratch_types=[
            pltpu.VMEM((ZROWS, ACC_W), jnp.float32),
            pltpu.VMEM((CHUNK, D), jnp.float32),
            pltpu.VMEM((CHUNK, D), jnp.float32),
            pltpu.VMEM((CHUNK, ACC_W), jnp.float32),
            pltpu.VMEM((CHUNK, DH), jnp.float32),
            pltpu.VMEM((CHUNK, DH), jnp.float32),
            pltpu.VMEM((CHUNK, DH), jnp.float32),
            pltpu.VMEM((CHUNK, DH), jnp.float32),
            pltpu.VMEM((CHUNK,), jnp.int32),
            pltpu.VMEM((CHUNK,), jnp.int32),
            pltpu.VMEM((CHUNK,), jnp.int32),
            pltpu.VMEM((CHUNK,), jnp.int32),
            pltpu.VMEM((CHUNK,), jnp.int32),
            pltpu.VMEM((CHUNK,), jnp.int32),
            pltpu.VMEM((CHUNK,), jnp.int32),
            pltpu.VMEM((CHUNK,), jnp.int32),
            pltpu.VMEM((CHUNK,), jnp.int32),
            pltpu.VMEM((CHUNK,), jnp.int32),
            pltpu.VMEM((CHUNK,), jnp.int32),
            pltpu.VMEM((CHUNK,), jnp.int32),
            pltpu.VMEM_SHARED((ACC_ROWS, ACC_W), jnp.float32),
            pltpu.SemaphoreType.DMA,
            pltpu.SemaphoreType.DMA,
            pltpu.SemaphoreType.DMA,
            pltpu.SemaphoreType.DMA,
            pltpu.SemaphoreType.DMA,
            pltpu.SemaphoreType.DMA,
        ],
    )(_sc_edge_kernel)
    return run(h_cat, table_cat, src1d, dst1d)


def _norm_body(rt_ref, raw_ref, out_ref):
    raw = raw_ref[...]
    msg = raw[:, 0:D]
    den = raw[:, D:D + NH]
    drep = jnp.dot(den, rt_ref[...], preferred_element_type=jnp.float32)
    out_ref[...] = jnp.maximum(msg / (drep + 1e-16), 0.0)


def _normalize(raw, rt):
    blk = 1000
    grid = (2 * N_NODE // blk,)
    return pl.pallas_call(
        _norm_body,
        grid=grid,
        in_specs=[
            pl.BlockSpec((NH, D), lambda i: (0, 0)),
            pl.BlockSpec((blk, ACC_W), lambda i: (i, 0)),
        ],
        out_specs=pl.BlockSpec((blk, D), lambda i: (i, 0)),
        out_shape=jax.ShapeDtypeStruct((2 * N_NODE, D), jnp.float32),
    )(rt, raw)


def kernel(x_user, x_item, edge_index_ut, edge_index_tu, W_user, b_user,
           W_item, b_item, att_src_ut, att_dst_ut, att_src_tu, att_dst_tu,
           k_W, k_b, q):
    f32 = jnp.float32
    rep = jnp.repeat(jnp.eye(NH, dtype=f32), DH, axis=0)
    z8 = jnp.zeros((D, NH), f32)

    def att_cols(att):
        return rep * att.reshape(D)[:, None]

    A_user = jnp.concatenate(
        [att_cols(att_src_ut), z8, att_cols(att_dst_tu), z8], axis=1)
    A_item = jnp.concatenate(
        [att_cols(att_src_tu), z8, att_cols(att_dst_ut), z8], axis=1)

    h_u, al_u = _project(x_user, W_user, b_user, A_user)
    h_i, al_i = _project(x_item, W_item, b_item, A_item)

    h_cat = jnp.concatenate([h_i, h_u], axis=0)
    table_tu = jnp.concatenate([al_i[:, 0:16], al_u[:, 16:32]], axis=0)
    table_ut = jnp.concatenate([al_u[:, 0:16], al_i[:, 16:32]], axis=0)
    table_cat = jnp.concatenate([table_tu, table_ut], axis=0)

    ei_tu = jnp.asarray(edge_index_tu, jnp.int32)
    ei_ut = jnp.asarray(edge_index_ut, jnp.int32)
    src1d = jnp.concatenate([ei_tu[0], ei_ut[0]])
    dst1d = jnp.concatenate([ei_tu[1], ei_ut[1]])

    raw = _sc_edge_phase(h_cat, table_cat, src1d, dst1d)
    return _normalize(raw, rep.T)


# speedup vs baseline: 68.4163x; 1.0006x over previous
import functools

import jax
import jax.numpy as jnp
from jax import lax
from jax.experimental import pallas as pl
from jax.experimental.pallas import tpu as pltpu
from jax.experimental.pallas import tpu_sc as plsc

N_NODE = 10000
N_EDGE = 320000
D = 128
NH = 8
DH = 16
ACC_W = 144

N_SUBCORES = 16
EDGES_PER_TILE = N_EDGE // N_SUBCORES
CHUNK = 80
N_CHUNKS = EDGES_PER_TILE // CHUNK
HALF = 5008
DUMMY = HALF
ACC_ROWS = 5024
ZROWS = 112
TBL_ROWS_PER_TILE = 2 * N_NODE // N_SUBCORES
SWEEP_ROWS = (HALF, N_NODE - HALF)


def _proj_body(x_ref, w_ref, b_ref, a_ref, h_ref, al_ref):
    h = jnp.dot(x_ref[...], w_ref[...], preferred_element_type=jnp.float32)
    h = h + b_ref[...]
    h_ref[...] = h
    al_ref[...] = jnp.dot(h, a_ref[...], preferred_element_type=jnp.float32)


def _project(x, w, b, a):
    blk = 1000
    grid = (N_NODE // blk,)
    return pl.pallas_call(
        _proj_body,
        grid=grid,
        in_specs=[
            pl.BlockSpec((blk, D), lambda i: (i, 0)),
            pl.BlockSpec((D, D), lambda i: (0, 0)),
            pl.BlockSpec((1, D), lambda i: (0, 0)),
            pl.BlockSpec((D, 32), lambda i: (0, 0)),
        ],
        out_specs=[
            pl.BlockSpec((blk, D), lambda i: (i, 0)),
            pl.BlockSpec((blk, 32), lambda i: (i, 0)),
        ],
        out_shape=[
            jax.ShapeDtypeStruct((N_NODE, D), jnp.float32),
            jax.ShapeDtypeStruct((N_NODE, 32), jnp.float32),
        ],
    )(x, w, b.reshape(1, D), a)


def _sc_edge_kernel(h_cat, table_cat, src1d, dst1d, raw,
                    zbuf_v, h_v0, h_v1, msg_v, as_v0, as_v1,
                    ad_v0, ad_v1,
                    idx_src0, idx_src1, idx_dst0, idx_dst1, idx_srcp0,
                    idx_srcp1, idx_dstd0, idx_dstd1, idx_dstt0, idx_dstt1,
                    idx_srct0, idx_srct1,
                    acc_sp, sem_a0, sem_a1, sem_b0, sem_b1,
                    sem_c0, sem_c1):
    h_v = (h_v0, h_v1)
    as_v = (as_v0, as_v1)
    ad_v = (ad_v0, ad_v1)
    idx_src = (idx_src0, idx_src1)
    idx_dst = (idx_dst0, idx_dst1)
    idx_srcp = (idx_srcp0, idx_srcp1)
    idx_dstd = (idx_dstd0, idx_dstd1)
    idx_dstt = (idx_dstt0, idx_dstt1)
    idx_srct = (idx_srct0, idx_srct1)
    sem_a = (sem_a0, sem_a1)
    sem_b = (sem_b0, sem_b1)
    sem_c = (sem_c0, sem_c1)
    c = lax.axis_index("c")
    s = lax.axis_index("s")

    zero16 = jnp.zeros((DH,), jnp.float32)
    head_idx = [jnp.full((DH,), h, jnp.int32) for h in range(NH)]

    def _zero_row(r, _):
        for j in range(ACC_W // DH):
            zbuf_v[r, pl.ds(j * DH, DH)] = zero16
        return 0
    lax.fori_loop(0, ZROWS, _zero_row, 0)

    ebase0 = c * N_EDGE + s * EDGES_PER_TILE
    c10k = c * N_NODE
    ctbl = c * 2 * N_NODE

    for p in range(2):
        z0 = s * (ACC_ROWS // N_SUBCORES)
        for zr, zn in ((0, ZROWS), (ZROWS, ZROWS), (2 * ZROWS, 90)):
            pltpu.sync_copy(zbuf_v.at[pl.ds(0, zn)], acc_sp.at[pl.ds(z0 + zr, zn)])
        plsc.subcore_barrier()

        lo = p * HALF

        def _stage(ch, b):
            ebase = ebase0 + ch * CHUNK
            pltpu.sync_copy(src1d.at[pl.ds(ebase, CHUNK)], idx_src[b])
            pltpu.sync_copy(dst1d.at[pl.ds(ebase, CHUNK)], idx_dst[b])
            for j in range(CHUNK // DH):
                sl = pl.ds(j * DH, DH)
                idx_srcp[b][sl] = idx_src[b][sl] + c10k
                idx_srct[b][sl] = idx_src[b][sl] + ctbl
                idx_dstt[b][sl] = idx_dst[b][sl] + (ctbl + N_NODE)
                t = idx_dst[b][sl] - lo
                if p == 0:
                    idx_dstd[b][sl] = jnp.minimum(t, DUMMY)
                else:
                    idx_dstd[b][sl] = jnp.where(t >= 0, t, DUMMY)
            return (pltpu.async_copy(table_cat.at[idx_srct[b]], as_v[b], sem_a[b]),
                    pltpu.async_copy(table_cat.at[idx_dstt[b]], ad_v[b], sem_b[b]),
                    pltpu.async_copy(h_cat.at[idx_srcp[b]], h_v[b], sem_c[b]))

        def _process(b, dmas):
            for d in dmas:
                d.wait()

            @plsc.parallel_loop(0, CHUNK, unroll=8)
            def _edge(e):
                a = as_v[b][e, :] + ad_v[b][e, :]
                a = jnp.maximum(a, a * 0.2)
                ev = jnp.exp(a)
                msg_v[e, pl.ds(D, DH)] = ev
                for h in range(NH):
                    ebc = ev.at[head_idx[h]].get(mode='promise_in_bounds')
                    sl = pl.ds(h * DH, DH)
                    msg_v[e, sl] = h_v[b][e, sl] * ebc

            pltpu.sync_copy(msg_v, acc_sp.at[idx_dstd[b]], add=True)

        def _chunk2(ch2, _):
            d0 = _stage(2 * ch2, 0)
            d1 = _stage(2 * ch2 + 1, 1)
            _process(0, d0)
            _process(1, d1)
            return 0
        lax.fori_loop(0, N_CHUNKS // 2, _chunk2, 0)
        plsc.subcore_barrier()

        rp = SWEEP_ROWS[p] // N_SUBCORES
        r0 = s * rp
        pltpu.sync_copy(acc_sp.at[pl.ds(r0, rp)],
                        raw.at[pl.ds(c * N_NODE + lo + r0, rp)])
        plsc.subcore_barrier()


def _sc_edge_phase(h_cat, table_cat, src1d, dst1d):
    mesh = plsc.VectorSubcoreMesh(core_axis_name="c", subcore_axis_name="s")
    run = functools.partial(
        pl.kernel,
        mesh=mesh,
        compiler_params=pltpu.CompilerParams(use_tc_tiling_on_sc=False,
                                             needs_layout_passes=False),
        out_type=jax.ShapeDtypeStruct((2 * N_NODE, ACC_W), jnp.float32),
        scratch_types=[
            pltpu.VMEM((ZROWS, ACC_W), jnp.float32),
            pltpu.VMEM((CHUNK, D), jnp.float32),
            pltpu.VMEM((CHUNK, D), jnp.float32),
            pltpu.VMEM((CHUNK, ACC_W), jnp.float32),
            pltpu.VMEM((CHUNK, DH), jnp.float32),
            pltpu.VMEM((CHUNK, DH), jnp.float32),
            pltpu.VMEM((CHUNK, DH), jnp.float32),
            pltpu.VMEM((CHUNK, DH), jnp.float32),
            pltpu.VMEM((CHUNK,), jnp.int32),
            pltpu.VMEM((CHUNK,), jnp.int32),
            pltpu.VMEM((CHUNK,), jnp.int32),
            pltpu.VMEM((CHUNK,), jnp.int32),
            pltpu.VMEM((CHUNK,), jnp.int32),
            pltpu.VMEM((CHUNK,), jnp.int32),
            pltpu.VMEM((CHUNK,), jnp.int32),
            pltpu.VMEM((CHUNK,), jnp.int32),
            pltpu.VMEM((CHUNK,), jnp.int32),
            pltpu.VMEM((CHUNK,), jnp.int32),
            pltpu.VMEM((CHUNK,), jnp.int32),
            pltpu.VMEM((CHUNK,), jnp.int32),
            pltpu.VMEM_SHARED((ACC_ROWS, ACC_W), jnp.float32),
            pltpu.SemaphoreType.DMA,
            pltpu.SemaphoreType.DMA,
            pltpu.SemaphoreType.DMA,
            pltpu.SemaphoreType.DMA,
            pltpu.SemaphoreType.DMA,
            pltpu.SemaphoreType.DMA,
        ],
    )(_sc_edge_kernel)
    return run(h_cat, table_cat, src1d, dst1d)


def _norm_body(rt_ref, raw_ref, out_ref):
    raw = raw_ref[...]
    msg = raw[:, 0:D]
    den = raw[:, D:D + NH]
    drep = jnp.dot(den, rt_ref[...], preferred_element_type=jnp.float32)
    out_ref[...] = jnp.maximum(msg / (drep + 1e-16), 0.0)


def _normalize(raw, rt):
    blk = 1000
    grid = (2 * N_NODE // blk,)
    return pl.pallas_call(
        _norm_body,
        grid=grid,
        in_specs=[
            pl.BlockSpec((NH, D), lambda i: (0, 0)),
            pl.BlockSpec((blk, ACC_W), lambda i: (i, 0)),
        ],
        out_specs=pl.BlockSpec((blk, D), lambda i: (i, 0)),
        out_shape=jax.ShapeDtypeStruct((2 * N_NODE, D), jnp.float32),
    )(rt, raw)


def kernel(x_user, x_item, edge_index_ut, edge_index_tu, W_user, b_user,
           W_item, b_item, att_src_ut, att_dst_ut, att_src_tu, att_dst_tu,
           k_W, k_b, q):
    f32 = jnp.float32
    rep = jnp.repeat(jnp.eye(NH, dtype=f32), DH, axis=0)
    z8 = jnp.zeros((D, NH), f32)

    def att_cols(att):
        return rep * att.reshape(D)[:, None]

    A_user = jnp.concatenate(
        [att_cols(att_src_ut), z8, att_cols(att_dst_tu), z8], axis=1)
    A_item = jnp.concatenate(
        [att_cols(att_src_tu), z8, att_cols(att_dst_ut), z8], axis=1)

    h_u, al_u = _project(x_user, W_user, b_user, A_user)
    h_i, al_i = _project(x_item, W_item, b_item, A_item)

    h_cat = jnp.concatenate([h_i, h_u], axis=0)
    table_tu = jnp.concatenate([al_i[:, 0:16], al_u[:, 16:32]], axis=0)
    table_ut = jnp.concatenate([al_u[:, 0:16], al_i[:, 16:32]], axis=0)
    table_cat = jnp.concatenate([table_tu, table_ut], axis=0)

    ei_tu = jnp.asarray(edge_index_tu, jnp.int32)
    ei_ut = jnp.asarray(edge_index_ut, jnp.int32)
    src1d = jnp.concatenate([ei_tu[0], ei_ut[0]])
    dst1d = jnp.concatenate([ei_tu[1], ei_ut[1]])

    raw = _sc_edge_phase(h_cat, table_cat, src1d, dst1d)
    return _normalize(raw, rep.T)
